# Initial kernel scaffold; baseline (speedup 1.0000x reference)
import jax
import jax.numpy as jnp
from jax.experimental import pallas as pl


def kernel(x):
    raise NotImplementedError("write your pallas kernel here")


# speedup vs baseline: 10.3595x; 10.3595x over previous
import functools

import jax
import jax.numpy as jnp
from jax import lax
from jax.experimental import pallas as pl
from jax.experimental.pallas import tpu as pltpu
from jax.experimental.pallas import tpu_sc as plsc

B = 64
N = 2048
D = 3
S = 512
L = 16
CHUNKS = N // L
UNROLL = 4
NUM_CORES = 2
NUM_SUBCORES = 16
NW = NUM_CORES * NUM_SUBCORES
PER_W = B // NW


def _fps_one_cloud(xv, dist, samp, idxv):
  inf_v = jnp.full((L,), jnp.inf, dtype=jnp.float32)
  lanes = lax.broadcasted_iota(jnp.int32, (L,), 0)
  lane0 = lanes == 0
  zero_i = jnp.zeros((L,), dtype=jnp.int32)
  neg_inf_v = jnp.full((L,), -jnp.inf, dtype=jnp.float32)
  int_max_v = jnp.full((L,), jnp.int32(2147483647), dtype=jnp.int32)

  def init_body(c, carry):
    dist[pl.ds(c * L, L)] = inf_v
    return carry

  lax.fori_loop(0, CHUNKS, init_body, 0)

  def pick(j_vec, i_vec):
    plsc.store_scatter(idxv, [i_vec], j_vec, mask=lane0)
    qx = plsc.load_gather(xv, [j_vec])
    qy = plsc.load_gather(xv, [j_vec + N])
    qz = plsc.load_gather(xv, [j_vec + 2 * N])
    plsc.store_scatter(samp, [i_vec], qx, mask=lane0)
    plsc.store_scatter(samp, [i_vec + S], qy, mask=lane0)
    plsc.store_scatter(samp, [i_vec + 2 * S], qz, mask=lane0)
    return qx, qy, qz

  px = zero_i.astype(jnp.float32) + xv[pl.ds(0, L)][0]
  py = zero_i.astype(jnp.float32) + xv[pl.ds(N, L)][0]
  pz = zero_i.astype(jnp.float32) + xv[pl.ds(2 * N, L)][0]
  plsc.store_scatter(idxv, [zero_i], zero_i, mask=lane0)
  plsc.store_scatter(samp, [zero_i], px, mask=lane0)
  plsc.store_scatter(samp, [zero_i + S], py, mask=lane0)
  plsc.store_scatter(samp, [zero_i + 2 * S], pz, mask=lane0)

  def iter_body(i, carry):
    px, py, pz = carry

    def chunk_body(c, c_carry):
      bv, bi = c_carry
      for u in range(UNROLL):
        off = (c * UNROLL + u) * L
        dx = xv[pl.ds(off, L)] - px
        dy = xv[pl.ds(off + N, L)] - py
        dz = xv[pl.ds(off + 2 * N, L)] - pz
        d = dx * dx + dy * dy + dz * dz
        dm = jnp.minimum(dist[pl.ds(off, L)], d)
        dist[pl.ds(off, L)] = dm
        pred = dm > bv
        bv = jnp.where(pred, dm, bv)
        bi = jnp.where(pred, off + lanes, bi)
      return (bv, bi)

    bv, bi = lax.fori_loop(0, CHUNKS // UNROLL, chunk_body,
                           (neg_inf_v, zero_i))
    m = jnp.max(bv)
    cand = jnp.where(bv == m, bi, int_max_v)
    j_vec = zero_i + jnp.min(cand)
    return pick(j_vec, zero_i + i)

  lax.fori_loop(1, S, iter_body, (px, py, pz))


@functools.partial(
    pl.kernel,
    mesh=plsc.VectorSubcoreMesh(core_axis_name="c", subcore_axis_name="s"),
    compiler_params=pltpu.CompilerParams(needs_layout_passes=False),
    out_type=[
        jax.ShapeDtypeStruct((B, D * S), jnp.float32),
        jax.ShapeDtypeStruct((B, S), jnp.int32),
    ],
    scratch_types=[
        pltpu.VMEM((D * N,), jnp.float32),
        pltpu.VMEM((N,), jnp.float32),
        pltpu.VMEM((D * S,), jnp.float32),
        pltpu.VMEM((S,), jnp.int32),
    ],
)
def _fps_sc(x_hbm, samp_hbm, idx_hbm, xv, dist, samp, idxv):
  wid = lax.axis_index("s") * NUM_CORES + lax.axis_index("c")
  for k in range(PER_W):
    b = wid * PER_W + k
    pltpu.sync_copy(x_hbm.at[b], xv)
    _fps_one_cloud(xv, dist, samp, idxv)
    pltpu.sync_copy(samp, samp_hbm.at[b])
    pltpu.sync_copy(idxv, idx_hbm.at[b])


@jax.jit
def kernel(x):
  xt = jnp.swapaxes(x, 1, 2).reshape(B, D * N)
  samp_t, idx = _fps_sc(xt)
  sampled = jnp.swapaxes(samp_t.reshape(B, D, S), 1, 2)
  return sampled, idx


# speedup vs baseline: 26.3788x; 2.5463x over previous
import functools

import jax
import jax.numpy as jnp
from jax import lax
from jax.experimental import pallas as pl
from jax.experimental.pallas import tpu as pltpu
from jax.experimental.pallas import tpu_sc as plsc

B = 64
N = 2048
D = 3
S = 512
L = 16
CHUNKS = N // L
UNROLL = 8
NUM_CORES = 2
NUM_SUBCORES = 16
NW = NUM_CORES * NUM_SUBCORES
PER_W = B // NW


def _fps_one_cloud(xv, dist, samp, idxv):
  inf_v = jnp.full((L,), jnp.inf, dtype=jnp.float32)
  lanes = lax.broadcasted_iota(jnp.int32, (L,), 0)
  lane0 = lanes == 0
  zero_i = jnp.zeros((L,), dtype=jnp.int32)
  neg_inf_v = jnp.full((L,), -jnp.inf, dtype=jnp.float32)
  int_max_v = jnp.full((L,), jnp.int32(2147483647), dtype=jnp.int32)

  def init_body(c, carry):
    dist[pl.ds(c * L, L)] = inf_v
    return carry

  lax.fori_loop(0, CHUNKS, init_body, 0)

  def pick(j_vec, i_vec):
    plsc.store_scatter(idxv, [i_vec], j_vec, mask=lane0)
    qx = plsc.load_gather(xv, [j_vec])
    qy = plsc.load_gather(xv, [j_vec + N])
    qz = plsc.load_gather(xv, [j_vec + 2 * N])
    plsc.store_scatter(samp, [i_vec], qx, mask=lane0)
    plsc.store_scatter(samp, [i_vec + S], qy, mask=lane0)
    plsc.store_scatter(samp, [i_vec + 2 * S], qz, mask=lane0)
    return qx, qy, qz

  px = zero_i.astype(jnp.float32) + xv[pl.ds(0, L)][0]
  py = zero_i.astype(jnp.float32) + xv[pl.ds(N, L)][0]
  pz = zero_i.astype(jnp.float32) + xv[pl.ds(2 * N, L)][0]
  plsc.store_scatter(idxv, [zero_i], zero_i, mask=lane0)
  plsc.store_scatter(samp, [zero_i], px, mask=lane0)
  plsc.store_scatter(samp, [zero_i + S], py, mask=lane0)
  plsc.store_scatter(samp, [zero_i + 2 * S], pz, mask=lane0)

  def iter_body(i, carry):
    px, py, pz = carry

    @plsc.parallel_loop(0, CHUNKS, 1, unroll=UNROLL,
                        carry=(neg_inf_v, zero_i))
    def chunk_body(c, c_carry):
      bv, bi = c_carry
      off = c * L
      dx = xv[pl.ds(off, L)] - px
      dy = xv[pl.ds(off + N, L)] - py
      dz = xv[pl.ds(off + 2 * N, L)] - pz
      d = dx * dx + (dy * dy + dz * dz)
      dm = jnp.minimum(dist[pl.ds(off, L)], d)
      dist[pl.ds(off, L)] = dm
      idx = off + lanes
      better = (dm > bv) | ((dm == bv) & (idx < bi))
      bv = jnp.where(better, dm, bv)
      bi = jnp.where(better, idx, bi)
      return (bv, bi)

    bv, bi = chunk_body
    m = jnp.max(bv)
    cand = jnp.where(bv == m, bi, int_max_v)
    j_vec = zero_i + jnp.min(cand)
    return pick(j_vec, zero_i + i)

  lax.fori_loop(1, S, iter_body, (px, py, pz))


@functools.partial(
    pl.kernel,
    mesh=plsc.VectorSubcoreMesh(core_axis_name="c", subcore_axis_name="s"),
    compiler_params=pltpu.CompilerParams(needs_layout_passes=False),
    out_type=[
        jax.ShapeDtypeStruct((B, D * S), jnp.float32),
        jax.ShapeDtypeStruct((B, S), jnp.int32),
    ],
    scratch_types=[
        pltpu.VMEM((D * N,), jnp.float32),
        pltpu.VMEM((N,), jnp.float32),
        pltpu.VMEM((D * S,), jnp.float32),
        pltpu.VMEM((S,), jnp.int32),
    ],
)
def _fps_sc(x_hbm, samp_hbm, idx_hbm, xv, dist, samp, idxv):
  wid = lax.axis_index("s") * NUM_CORES + lax.axis_index("c")
  for k in range(PER_W):
    b = wid * PER_W + k
    pltpu.sync_copy(x_hbm.at[b], xv)
    _fps_one_cloud(xv, dist, samp, idxv)
    pltpu.sync_copy(samp, samp_hbm.at[b])
    pltpu.sync_copy(idxv, idx_hbm.at[b])


@jax.jit
def kernel(x):
  xt = jnp.swapaxes(x, 1, 2).reshape(B, D * N)
  samp_t, idx = _fps_sc(xt)
  sampled = jnp.swapaxes(samp_t.reshape(B, D, S), 1, 2)
  return sampled, idx
